# baseline (speedup 1.0000x reference)
import functools

import jax
import jax.numpy as jnp
from jax import lax
from jax.experimental import pallas as pl
from jax.experimental.pallas import tpu as pltpu
from jax.experimental.pallas import tpu_sc as plsc

B = 64
L = 128
K = 8192
CHUNK = 8
SG = 8
NSG = K // (CHUNK * SG)
NQ = B * L
OCT = 8


def _phase1_body(z8_ref, cbt_ref, accd_ref, accg_ref):
    for oct_i in range(B // OCT):
        zbb = [z8_ref[oct_i * OCT + i] for i in range(OCT)]

        def step(sg, carry):
            acc_d = list(carry[:OCT])
            acc_g = list(carry[OCT:])
            chunks = [
                cbt_ref[pl.ds((sg * SG + c) * CHUNK, CHUNK), :] for c in range(SG)
            ]
            for i in range(OCT):
                m0 = jnp.abs(zbb[i] - chunks[0])
                m1 = jnp.abs(zbb[i] - chunks[1])
                for c in range(2, SG, 2):
                    m0 = jnp.minimum(m0, jnp.abs(zbb[i] - chunks[c]))
                    m1 = jnp.minimum(m1, jnp.abs(zbb[i] - chunks[c + 1]))
                m = jnp.minimum(m0, m1)
                pred = m < acc_d[i]
                acc_d[i] = jnp.minimum(acc_d[i], m)
                acc_g[i] = jnp.where(pred, sg, acc_g[i])
            return tuple(acc_d) + tuple(acc_g)

        init = tuple(jnp.full((CHUNK, L), jnp.inf, dtype=jnp.float32) for _ in range(OCT)) + tuple(
            jnp.zeros((CHUNK, L), dtype=jnp.int32) for _ in range(OCT)
        )
        res = jax.lax.fori_loop(0, NSG, step, init, unroll=2)

        for i in range(OCT):
            b = oct_i * OCT + i
            accd_ref[b] = res[i]
            accg_ref[b] = res[OCT + i]


def _phase1(z8, cbt):
    return pl.pallas_call(
        _phase1_body,
        out_shape=(
            jax.ShapeDtypeStruct((B, CHUNK, L), jnp.float32),
            jax.ShapeDtypeStruct((B, CHUNK, L), jnp.int32),
        ),
    )(z8, cbt)


_NC = 2
_NS = 16
_NW = _NC * _NS
_QPW = NQ // _NW
_HALF = _QPW // 2
_BPW = _QPW // L


def _refine_body(t2_hbm, z_hbm, accd_hbm, accg_hbm, idx_hbm, zq_hbm, part_hbm,
                 av, gv, zv, dv, kbv, rvall, cand, idxv, zqv, lossv, sem):
    wid = lax.axis_index("s") * _NC + lax.axis_index("c")
    base = wid * _QPW
    b0 = wid * _BPW

    for bb in range(_BPW):
        pltpu.sync_copy(accd_hbm.at[b0 + bb], av.at[pl.ds(bb * CHUNK, CHUNK)])
        pltpu.sync_copy(accg_hbm.at[b0 + bb], gv.at[pl.ds(bb * CHUNK, CHUNK)])
    pltpu.sync_copy(z_hbm.at[pl.ds(base, _QPW)], zv)

    lane16 = lax.iota(jnp.int32, 16)
    for bb in range(_BPW):
        for t in range(L // 16):
            l16 = lane16 + t * 16
            best_d = av[bb * CHUNK, pl.ds(t * 16, 16)]
            best_g = gv[bb * CHUNK, pl.ds(t * 16, 16)]
            best_s = jnp.zeros((16,), dtype=jnp.int32)
            for s in range(1, CHUNK):
                d_s = av[bb * CHUNK + s, pl.ds(t * 16, 16)]
                g_s = gv[bb * CHUNK + s, pl.ds(t * 16, 16)]
                take = (d_s < best_d) | ((d_s == best_d) & (g_s < best_g))
                best_d = jnp.where(take, d_s, best_d)
                best_g = jnp.where(take, g_s, best_g)
                best_s = jnp.where(take, s, best_s)
            q0 = bb * L + t * 16
            dv[pl.ds(q0, 16)] = best_d
            kbv[pl.ds(q0, 16)] = best_g * (SG * CHUNK) + best_s
            r2 = (best_g * CHUNK + best_s) * L + l16
            c16 = q0 // 16
            h, cc = divmod(c16, _HALF // 16)
            for c in range(SG):
                rvall[h * SG + c, pl.ds(cc * 16, 16)] = r2 * SG + c

    copies = []
    for c in range(SG):
        for h in range(2):
            copies.append(pltpu.async_copy(
                t2_hbm.at[rvall.at[h * SG + c]],
                cand.at[c, pl.ds(h * _HALF, _HALF)],
                sem,
            ))
    for cp in copies:
        cp.wait()

    loss16 = jnp.zeros((16,), dtype=jnp.float32)
    for t in range(_QPW // 16):
        q16 = pl.ds(t * 16, 16)
        z16 = zv[q16]
        d16 = dv[q16]
        kb16 = kbv[q16]
        best_c = jnp.zeros((16,), dtype=jnp.int32)
        best_v = jnp.zeros((16,), dtype=jnp.float32)
        for c in range(SG - 1, -1, -1):
            cj = cand[c, q16]
            hit = jnp.abs(z16 - cj) == d16
            best_c = jnp.where(hit, c, best_c)
            best_v = jnp.where(hit, cj, best_v)
        idxv[q16] = kb16 + best_c * CHUNK
        zqv[q16] = best_v
        loss16 = loss16 + d16 * d16

    lossv[pl.ds(0, 16)] = loss16
    pltpu.sync_copy(idxv, idx_hbm.at[pl.ds(base, _QPW)])
    pltpu.sync_copy(zqv, zq_hbm.at[pl.ds(base, _QPW)])
    pltpu.sync_copy(lossv, part_hbm.at[wid])


def _refine(t2_flat, z_flat, accd, accg):
    mesh = plsc.VectorSubcoreMesh(core_axis_name="c", subcore_axis_name="s")
    kern = functools.partial(
        pl.kernel,
        mesh=mesh,
        out_type=(
            jax.ShapeDtypeStruct((NQ,), jnp.int32),
            jax.ShapeDtypeStruct((NQ,), jnp.float32),
            jax.ShapeDtypeStruct((_NW, 16), jnp.float32),
        ),
        scratch_types=[
            pltpu.VMEM((_BPW * CHUNK, L), jnp.float32),
            pltpu.VMEM((_BPW * CHUNK, L), jnp.int32),
            pltpu.VMEM((_QPW,), jnp.float32),
            pltpu.VMEM((_QPW,), jnp.float32),
            pltpu.VMEM((_QPW,), jnp.int32),
            pltpu.VMEM((2 * SG, _HALF), jnp.int32),
            pltpu.VMEM((SG, _QPW), jnp.float32),
            pltpu.VMEM((_QPW,), jnp.int32),
            pltpu.VMEM((_QPW,), jnp.float32),
            pltpu.VMEM((16,), jnp.float32),
            pltpu.SemaphoreType.DMA,
        ],
    )(_refine_body)
    return kern(t2_flat, z_flat, accd, accg)


def kernel(z_batch, codebook, iter):
    cbt = codebook.T
    z8 = jnp.broadcast_to(z_batch[:, None, :], (B, CHUNK, L))
    t2_flat = (
        codebook.reshape(L, NSG, SG, CHUNK).transpose(1, 3, 0, 2).reshape(L * K)
    )
    accd, accg = _phase1(z8, cbt)
    idx_flat, zq_flat, parts = _refine(t2_flat, z_batch.reshape(NQ), accd, accg)
    loss = jnp.sum(parts) * (1.25 / NQ)
    zq = zq_flat.reshape(B, L)
    idx = idx_flat.reshape(B, L)
    z_q_st = z_batch + jax.lax.stop_gradient(zq - z_batch)
    return (z_q_st, loss, idx)


# speedup vs baseline: 2.0611x; 2.0611x over previous
import functools

import jax
import jax.numpy as jnp
from jax import lax
from jax.experimental import pallas as pl
from jax.experimental.pallas import tpu as pltpu
from jax.experimental.pallas import tpu_sc as plsc

B = 64
L = 128
K = 8192
CHUNK = 8
SG = 8
NSG = K // (CHUNK * SG)
NQ = B * L
OCT = 8


def _phase1_body(z8_ref, cbt_ref, accd_ref, accg_ref):
    for oct_i in range(B // OCT):
        zbb = [z8_ref[oct_i * OCT + i] for i in range(OCT)]

        def step(sg, carry):
            acc_d = list(carry[:OCT])
            acc_g = list(carry[OCT:])
            chunks = [
                cbt_ref[pl.ds((sg * SG + c) * CHUNK, CHUNK), :] for c in range(SG)
            ]
            for i in range(OCT):
                m0 = jnp.abs(zbb[i] - chunks[0])
                m1 = jnp.abs(zbb[i] - chunks[1])
                for c in range(2, SG, 2):
                    m0 = jnp.minimum(m0, jnp.abs(zbb[i] - chunks[c]))
                    m1 = jnp.minimum(m1, jnp.abs(zbb[i] - chunks[c + 1]))
                m = jnp.minimum(m0, m1)
                pred = m < acc_d[i]
                acc_d[i] = jnp.minimum(acc_d[i], m)
                acc_g[i] = jnp.where(pred, sg, acc_g[i])
            return tuple(acc_d) + tuple(acc_g)

        init = tuple(jnp.full((CHUNK, L), jnp.inf, dtype=jnp.float32) for _ in range(OCT)) + tuple(
            jnp.zeros((CHUNK, L), dtype=jnp.int32) for _ in range(OCT)
        )
        res = jax.lax.fori_loop(0, NSG, step, init, unroll=2)

        for i in range(OCT):
            b = oct_i * OCT + i
            accd_ref[b] = res[i]
            accg_ref[b] = res[OCT + i]


def _phase1(z8, cbt):
    return pl.pallas_call(
        _phase1_body,
        out_shape=(
            jax.ShapeDtypeStruct((B, CHUNK, L), jnp.float32),
            jax.ShapeDtypeStruct((B, CHUNK, L), jnp.int32),
        ),
    )(z8, cbt)


_NC = 2
_NS = 16
_NW = _NC * _NS
_QPW = NQ // _NW
_HALF = _QPW // 2
_BPW = _QPW // L


def _refine_body(t2_hbm, z_hbm, accd_hbm, accg_hbm, idx_hbm, zq_hbm, part_hbm,
                 av, gv, zv, dv, kbv, rvall, cand, idxv, zqv, lossv, sem):
    wid = lax.axis_index("s") * _NC + lax.axis_index("c")
    base = wid * _QPW
    b0 = wid * _BPW

    for bb in range(_BPW):
        pltpu.sync_copy(accd_hbm.at[b0 + bb], av.at[pl.ds(bb * CHUNK, CHUNK)])
        pltpu.sync_copy(accg_hbm.at[b0 + bb], gv.at[pl.ds(bb * CHUNK, CHUNK)])
    pltpu.sync_copy(z_hbm.at[pl.ds(base, _QPW)], zv)

    lane16 = lax.iota(jnp.int32, 16)
    for bb in range(_BPW):
        for t in range(L // 16):
            l16 = lane16 + t * 16
            best_d = av[bb * CHUNK, pl.ds(t * 16, 16)]
            best_g = gv[bb * CHUNK, pl.ds(t * 16, 16)]
            best_s = jnp.zeros((16,), dtype=jnp.int32)
            for s in range(1, CHUNK):
                d_s = av[bb * CHUNK + s, pl.ds(t * 16, 16)]
                g_s = gv[bb * CHUNK + s, pl.ds(t * 16, 16)]
                take = (d_s < best_d) | ((d_s == best_d) & (g_s < best_g))
                best_d = jnp.where(take, d_s, best_d)
                best_g = jnp.where(take, g_s, best_g)
                best_s = jnp.where(take, s, best_s)
            q0 = bb * L + t * 16
            dv[pl.ds(q0, 16)] = best_d
            kb = best_g * (SG * CHUNK) + best_s
            kbv[pl.ds(q0, 16)] = kb
            r2 = kb * L + l16
            c16 = q0 // 16
            h, cc = divmod(c16, _HALF // 16)
            for c in range(SG):
                rvall[h * SG + c, pl.ds(cc * 16, 16)] = r2 + c * (CHUNK * L)

    copies = []
    for c in range(SG):
        for h in range(2):
            copies.append(pltpu.async_copy(
                t2_hbm.at[rvall.at[h * SG + c]],
                cand.at[c, pl.ds(h * _HALF, _HALF)],
                sem,
            ))
    for cp in copies:
        cp.wait()

    loss16 = jnp.zeros((16,), dtype=jnp.float32)
    for t in range(_QPW // 16):
        q16 = pl.ds(t * 16, 16)
        z16 = zv[q16]
        d16 = dv[q16]
        kb16 = kbv[q16]
        best_c = jnp.zeros((16,), dtype=jnp.int32)
        best_v = jnp.zeros((16,), dtype=jnp.float32)
        for c in range(SG - 1, -1, -1):
            cj = cand[c, q16]
            hit = jnp.abs(z16 - cj) == d16
            best_c = jnp.where(hit, c, best_c)
            best_v = jnp.where(hit, cj, best_v)
        idxv[q16] = kb16 + best_c * CHUNK
        zqv[q16] = best_v
        loss16 = loss16 + d16 * d16

    lossv[pl.ds(0, 16)] = loss16
    pltpu.sync_copy(idxv, idx_hbm.at[pl.ds(base, _QPW)])
    pltpu.sync_copy(zqv, zq_hbm.at[pl.ds(base, _QPW)])
    pltpu.sync_copy(lossv, part_hbm.at[wid])


def _refine(t2_flat, z_flat, accd, accg):
    mesh = plsc.VectorSubcoreMesh(core_axis_name="c", subcore_axis_name="s")
    kern = functools.partial(
        pl.kernel,
        mesh=mesh,
        out_type=(
            jax.ShapeDtypeStruct((NQ,), jnp.int32),
            jax.ShapeDtypeStruct((NQ,), jnp.float32),
            jax.ShapeDtypeStruct((_NW, 16), jnp.float32),
        ),
        scratch_types=[
            pltpu.VMEM((_BPW * CHUNK, L), jnp.float32),
            pltpu.VMEM((_BPW * CHUNK, L), jnp.int32),
            pltpu.VMEM((_QPW,), jnp.float32),
            pltpu.VMEM((_QPW,), jnp.float32),
            pltpu.VMEM((_QPW,), jnp.int32),
            pltpu.VMEM((2 * SG, _HALF), jnp.int32),
            pltpu.VMEM((SG, _QPW), jnp.float32),
            pltpu.VMEM((_QPW,), jnp.int32),
            pltpu.VMEM((_QPW,), jnp.float32),
            pltpu.VMEM((16,), jnp.float32),
            pltpu.SemaphoreType.DMA,
        ],
    )(_refine_body)
    return kern(t2_flat, z_flat, accd, accg)


def kernel(z_batch, codebook, iter):
    cbt = codebook.T
    z8 = jnp.broadcast_to(z_batch[:, None, :], (B, CHUNK, L))
    accd, accg = _phase1(z8, cbt)
    idx_flat, zq_flat, parts = _refine(cbt.reshape(K * L), z_batch.reshape(NQ), accd, accg)
    loss = jnp.sum(parts) * (1.25 / NQ)
    zq = zq_flat.reshape(B, L)
    idx = idx_flat.reshape(B, L)
    z_q_st = z_batch + jax.lax.stop_gradient(zq - z_batch)
    return (z_q_st, loss, idx)
